# baseline (device time: 137502 ns/iter reference)
import jax
import jax.numpy as jnp
from jax import lax
from jax.experimental import pallas as pl
from jax.experimental.pallas import tpu as pltpu

N_DEV = 8
M = 1536
H_SH = 3072
CH = M // N_DEV
NB = 12
RB = M // NB
BWD = H_SH // NB


def kernel(x, Wg, Wu, Wd):
    xb = x.astype(jnp.bfloat16)

    def body(x_ref, wg_hbm, wu_hbm, wd_hbm, out_ref,
             wgb, wub, wdb, stage_g, stage_u, stage_d, dma_sems,
             recv_buf, rs_send_sems, rs_recv_sems,
             agr_send_sems, agr_recv_sems, agl_send_sems, agl_recv_sems):
        my = lax.axis_index("i")
        left = (my - 1) % N_DEV
        right = (my + 1) % N_DEV

        def srcs_at(b):
            return (wg_hbm.at[pl.ds(b * RB, RB), :],
                    wu_hbm.at[pl.ds(b * RB, RB), :],
                    wd_hbm.at[pl.ds(b * BWD, BWD), :])

        def start_block(b):
            sg, su, sd = srcs_at(b)
            pltpu.make_async_copy(sg, stage_g.at[b % 2], dma_sems.at[0, b % 2]).start()
            pltpu.make_async_copy(su, stage_u.at[b % 2], dma_sems.at[1, b % 2]).start()
            pltpu.make_async_copy(sd, stage_d.at[b % 2], dma_sems.at[2, b % 2]).start()

        my_row = pl.ds((my % N_DEV) * CH, CH)

        def stream_step(b, carry):
            g_acc, u_acc = carry
            @pl.when(b + 1 < NB)
            def _():
                start_block(b + 1)
            sg, su, sd = srcs_at(b)
            pltpu.make_async_copy(sg, stage_g.at[b % 2], dma_sems.at[0, b % 2]).wait()
            pltpu.make_async_copy(su, stage_u.at[b % 2], dma_sems.at[1, b % 2]).wait()
            pltpu.make_async_copy(sd, stage_d.at[b % 2], dma_sems.at[2, b % 2]).wait()
            wg_blk = stage_g[b % 2].astype(jnp.bfloat16)
            wu_blk = stage_u[b % 2].astype(jnp.bfloat16)
            wgb[pl.ds(b * RB, RB), :] = wg_blk
            wub[pl.ds(b * RB, RB), :] = wu_blk
            wdb[pl.ds(b * BWD, BWD), :] = stage_d[b % 2].astype(jnp.bfloat16)
            xk = x_ref[my_row, pl.ds(b * RB, RB)]
            g_acc = g_acc + jnp.dot(xk, wg_blk, preferred_element_type=jnp.float32)
            u_acc = u_acc + jnp.dot(xk, wu_blk, preferred_element_type=jnp.float32)
            return (g_acc, u_acc)

        start_block(0)
        g0, u0 = lax.fori_loop(
            0, NB, stream_step,
            (jnp.zeros((CH, H_SH), jnp.float32), jnp.zeros((CH, H_SH), jnp.float32)),
        )
        h0 = (g0 * (u0 * jax.nn.sigmoid(u0))).astype(jnp.bfloat16)
        out_ref[my_row, :] = jnp.dot(
            h0, wdb[...], preferred_element_type=jnp.float32
        ).astype(jnp.bfloat16)

        def compute_chunk(c):
            row = pl.ds(c * CH, CH)
            xv = x_ref[row, :]
            g = jnp.dot(xv, wgb[...], preferred_element_type=jnp.float32)
            u = jnp.dot(xv, wub[...], preferred_element_type=jnp.float32)
            h = (g * (u * jax.nn.sigmoid(u))).astype(jnp.bfloat16)
            out_ref[row, :] = jnp.dot(
                h, wdb[...], preferred_element_type=jnp.float32
            ).astype(jnp.bfloat16)

        barrier_sem = pltpu.get_barrier_semaphore()
        for nbr in (left, right):
            pl.semaphore_signal(barrier_sem, inc=1, device_id=(nbr,),
                                device_id_type=pl.DeviceIdType.MESH)
        pl.semaphore_wait(barrier_sem, 2)

        def rs_step(s, _):
            send_c = (my - s) % N_DEV
            rdma = pltpu.make_async_remote_copy(
                src_ref=out_ref.at[pl.ds(send_c * CH, CH), :],
                dst_ref=recv_buf.at[s % 2],
                send_sem=rs_send_sems.at[s],
                recv_sem=rs_recv_sems.at[s],
                device_id=(right,),
                device_id_type=pl.DeviceIdType.MESH,
            )
            rdma.start()
            rc = (my - s - 1) % N_DEV
            compute_chunk(rc)
            rdma.wait_recv()
            row = pl.ds(rc * CH, CH)
            out_ref[row, :] = out_ref[row, :] + recv_buf[s % 2]
            rdma.wait_send()
            return _

        lax.fori_loop(0, N_DEV - 1, rs_step, None)

        ag_rdmas = []
        for t in range(4):
            cr = (my + 1 - t) % N_DEV
            rowr = pl.ds(cr * CH, CH)
            r_rdma = pltpu.make_async_remote_copy(
                src_ref=out_ref.at[rowr, :],
                dst_ref=out_ref.at[rowr, :],
                send_sem=agr_send_sems.at[t],
                recv_sem=agr_recv_sems.at[t],
                device_id=(right,),
                device_id_type=pl.DeviceIdType.MESH,
            )
            r_rdma.start()
            ag_rdmas.append(r_rdma)
            l_rdma = None
            if t < 3:
                cl = (my + 1 + t) % N_DEV
                rowl = pl.ds(cl * CH, CH)
                l_rdma = pltpu.make_async_remote_copy(
                    src_ref=out_ref.at[rowl, :],
                    dst_ref=out_ref.at[rowl, :],
                    send_sem=agl_send_sems.at[t],
                    recv_sem=agl_recv_sems.at[t],
                    device_id=(left,),
                    device_id_type=pl.DeviceIdType.MESH,
                )
                l_rdma.start()
                ag_rdmas.append(l_rdma)
            r_rdma.wait_recv()
            if l_rdma is not None:
                l_rdma.wait_recv()
        for rdma in ag_rdmas:
            rdma.wait_send()

    return pl.pallas_call(
        body,
        out_shape=jax.ShapeDtypeStruct((M, M), jnp.bfloat16),
        in_specs=[
            pl.BlockSpec(memory_space=pltpu.VMEM),
            pl.BlockSpec(memory_space=pl.ANY),
            pl.BlockSpec(memory_space=pl.ANY),
            pl.BlockSpec(memory_space=pl.ANY),
        ],
        out_specs=pl.BlockSpec(memory_space=pltpu.VMEM),
        scratch_shapes=[
            pltpu.VMEM((M, H_SH), jnp.bfloat16),
            pltpu.VMEM((M, H_SH), jnp.bfloat16),
            pltpu.VMEM((H_SH, M), jnp.bfloat16),
            pltpu.VMEM((2, RB, H_SH), jnp.float32),
            pltpu.VMEM((2, RB, H_SH), jnp.float32),
            pltpu.VMEM((2, BWD, M), jnp.float32),
            pltpu.SemaphoreType.DMA((3, 2)),
            pltpu.VMEM((2, CH, M), jnp.bfloat16),
            pltpu.SemaphoreType.DMA((N_DEV - 1,)),
            pltpu.SemaphoreType.DMA((N_DEV - 1,)),
            pltpu.SemaphoreType.DMA((4,)),
            pltpu.SemaphoreType.DMA((4,)),
            pltpu.SemaphoreType.DMA((3,)),
            pltpu.SemaphoreType.DMA((3,)),
        ],
        compiler_params=pltpu.CompilerParams(
            collective_id=0,
            vmem_limit_bytes=63 * 1024 * 1024,
        ),
    )(xb, Wg, Wu, Wd)


# device time: 134683 ns/iter; 1.0209x vs baseline; 1.0209x over previous
import jax
import jax.numpy as jnp
from jax import lax
from jax.experimental import pallas as pl
from jax.experimental.pallas import tpu as pltpu

N_DEV = 8
M = 1536
H_SH = 3072
CH = M // N_DEV
NB = 8
BW = H_SH // NB


def kernel(x, Wg, Wu, Wd):
    xb = x.astype(jnp.bfloat16)

    def body(x_ref, wg_hbm, wu_hbm, wd_hbm, out_ref,
             wgb, wub, wdb, stage_g, stage_u, stage_d, dma_sems,
             recv_buf, rs_send_sems, rs_recv_sems,
             ag_send_sems, ag_recv_sems):
        my = lax.axis_index("i")
        left = (my - 1) % N_DEV
        right = (my + 1) % N_DEV

        def srcs_at(b):
            return (wg_hbm.at[:, pl.ds(b * BW, BW)],
                    wu_hbm.at[:, pl.ds(b * BW, BW)],
                    wd_hbm.at[pl.ds(b * BW, BW), :])

        def start_block(b):
            sg, su, sd = srcs_at(b)
            pltpu.make_async_copy(sg, stage_g.at[b % 2], dma_sems.at[0, b % 2]).start()
            pltpu.make_async_copy(su, stage_u.at[b % 2], dma_sems.at[1, b % 2]).start()
            pltpu.make_async_copy(sd, stage_d.at[b % 2], dma_sems.at[2, b % 2]).start()

        my_row = pl.ds((my % N_DEV) * CH, CH)
        xv_my = x_ref[my_row, :]

        def stream_step(b, acc):
            @pl.when(b + 1 < NB)
            def _():
                start_block(b + 1)
            sg, su, sd = srcs_at(b)
            pltpu.make_async_copy(sg, stage_g.at[b % 2], dma_sems.at[0, b % 2]).wait()
            pltpu.make_async_copy(su, stage_u.at[b % 2], dma_sems.at[1, b % 2]).wait()
            pltpu.make_async_copy(sd, stage_d.at[b % 2], dma_sems.at[2, b % 2]).wait()
            col = pl.ds(b * BW, BW)
            wg_blk = stage_g[b % 2].astype(jnp.bfloat16)
            wu_blk = stage_u[b % 2].astype(jnp.bfloat16)
            wd_blk = stage_d[b % 2].astype(jnp.bfloat16)
            wgb[:, col] = wg_blk
            wub[:, col] = wu_blk
            wdb[pl.ds(b * BW, BW), :] = wd_blk
            g = jnp.dot(xv_my, wg_blk, preferred_element_type=jnp.float32)
            u = jnp.dot(xv_my, wu_blk, preferred_element_type=jnp.float32)
            h = (g * (u * jax.nn.sigmoid(u))).astype(jnp.bfloat16)
            return acc + jnp.dot(h, wd_blk, preferred_element_type=jnp.float32)

        start_block(0)
        acc0 = lax.fori_loop(
            0, NB, stream_step, jnp.zeros((CH, M), jnp.float32)
        )
        out_ref[my_row, :] = acc0.astype(jnp.bfloat16)

        def compute_chunk(c):
            row = pl.ds(c * CH, CH)
            xv = x_ref[row, :]
            g = jnp.dot(xv, wgb[...], preferred_element_type=jnp.float32)
            u = jnp.dot(xv, wub[...], preferred_element_type=jnp.float32)
            h = (g * (u * jax.nn.sigmoid(u))).astype(jnp.bfloat16)
            out_ref[row, :] = jnp.dot(
                h, wdb[...], preferred_element_type=jnp.float32
            ).astype(jnp.bfloat16)

        barrier_sem = pltpu.get_barrier_semaphore()
        for nbr in (left, right):
            pl.semaphore_signal(barrier_sem, inc=1, device_id=(nbr,),
                                device_id_type=pl.DeviceIdType.MESH)
        pl.semaphore_wait(barrier_sem, 2)

        def rs_step(s, _):
            send_c = (my - s) % N_DEV
            rdma = pltpu.make_async_remote_copy(
                src_ref=out_ref.at[pl.ds(send_c * CH, CH), :],
                dst_ref=recv_buf.at[s % 2],
                send_sem=rs_send_sems.at[s],
                recv_sem=rs_recv_sems.at[s],
                device_id=(right,),
                device_id_type=pl.DeviceIdType.MESH,
            )
            rdma.start()
            rc = (my - s - 1) % N_DEV
            compute_chunk(rc)
            rdma.wait_recv()
            row = pl.ds(rc * CH, CH)
            out_ref[row, :] = out_ref[row, :] + recv_buf[s % 2]
            rdma.wait_send()
            return _

        lax.fori_loop(0, N_DEV - 1, rs_step, None)

        own = (my + 1) % N_DEV
        own_row = pl.ds(own * CH, CH)
        ag_rdmas = []
        for k in range(1, N_DEV):
            rdma = pltpu.make_async_remote_copy(
                src_ref=out_ref.at[own_row, :],
                dst_ref=out_ref.at[own_row, :],
                send_sem=ag_send_sems.at[k - 1],
                recv_sem=ag_recv_sems.at[k - 1],
                device_id=((my + k) % N_DEV,),
                device_id_type=pl.DeviceIdType.MESH,
            )
            rdma.start()
            ag_rdmas.append(rdma)
        for rdma in ag_rdmas:
            rdma.wait_recv()
        for rdma in ag_rdmas:
            rdma.wait_send()

    return pl.pallas_call(
        body,
        out_shape=jax.ShapeDtypeStruct((M, M), jnp.bfloat16),
        in_specs=[
            pl.BlockSpec(memory_space=pltpu.VMEM),
            pl.BlockSpec(memory_space=pl.ANY),
            pl.BlockSpec(memory_space=pl.ANY),
            pl.BlockSpec(memory_space=pl.ANY),
        ],
        out_specs=pl.BlockSpec(memory_space=pltpu.VMEM),
        scratch_shapes=[
            pltpu.VMEM((M, H_SH), jnp.bfloat16),
            pltpu.VMEM((M, H_SH), jnp.bfloat16),
            pltpu.VMEM((H_SH, M), jnp.bfloat16),
            pltpu.VMEM((2, M, BW), jnp.float32),
            pltpu.VMEM((2, M, BW), jnp.float32),
            pltpu.VMEM((2, BW, M), jnp.float32),
            pltpu.SemaphoreType.DMA((3, 2)),
            pltpu.VMEM((2, CH, M), jnp.bfloat16),
            pltpu.SemaphoreType.DMA((N_DEV - 1,)),
            pltpu.SemaphoreType.DMA((N_DEV - 1,)),
            pltpu.SemaphoreType.DMA((N_DEV - 1,)),
            pltpu.SemaphoreType.DMA((N_DEV - 1,)),
        ],
        compiler_params=pltpu.CompilerParams(
            collective_id=0,
            vmem_limit_bytes=63 * 1024 * 1024,
        ),
    )(xb, Wg, Wu, Wd)


# device time: 124910 ns/iter; 1.1008x vs baseline; 1.0782x over previous
import jax
import jax.numpy as jnp
from jax import lax
from jax.experimental import pallas as pl
from jax.experimental.pallas import tpu as pltpu

N_DEV = 8
M = 1536
H_SH = 3072
CH = M // N_DEV
NB = 8
BW = H_SH // NB


def kernel(x, Wg, Wu, Wd):
    xb = x.astype(jnp.bfloat16)

    def body(x_ref, wg_hbm, wu_hbm, wd_hbm, out_ref,
             wgb, wub, wdb, stage_g, stage_u, stage_d, dma_sems,
             recv_buf, rs_send_sems, rs_recv_sems,
             ag_send_sems, ag_recv_sems):
        my = lax.axis_index("i")
        left = (my - 1) % N_DEV
        right = (my + 1) % N_DEV

        def srcs_at(b):
            return (wg_hbm.at[:, pl.ds(b * BW, BW)],
                    wu_hbm.at[:, pl.ds(b * BW, BW)],
                    wd_hbm.at[pl.ds(b * BW, BW), :])

        def start_block(b):
            sg, su, sd = srcs_at(b)
            pltpu.make_async_copy(sg, stage_g.at[b % 2], dma_sems.at[0, b % 2]).start()
            pltpu.make_async_copy(su, stage_u.at[b % 2], dma_sems.at[1, b % 2]).start()
            pltpu.make_async_copy(sd, stage_d.at[b % 2], dma_sems.at[2, b % 2]).start()

        my_row = pl.ds((my % N_DEV) * CH, CH)
        xv_my = x_ref[my_row, :]

        def stream_step(b, acc):
            @pl.when(b + 1 < NB)
            def _():
                start_block(b + 1)
            sg, su, sd = srcs_at(b)
            pltpu.make_async_copy(sg, stage_g.at[b % 2], dma_sems.at[0, b % 2]).wait()
            pltpu.make_async_copy(su, stage_u.at[b % 2], dma_sems.at[1, b % 2]).wait()
            pltpu.make_async_copy(sd, stage_d.at[b % 2], dma_sems.at[2, b % 2]).wait()
            col = pl.ds(b * BW, BW)
            wg_blk = stage_g[b % 2].astype(jnp.bfloat16)
            wu_blk = stage_u[b % 2].astype(jnp.bfloat16)
            wd_blk = stage_d[b % 2].astype(jnp.bfloat16)
            wgb[:, col] = wg_blk
            wub[:, col] = wu_blk
            wdb[pl.ds(b * BW, BW), :] = wd_blk
            g = jnp.dot(xv_my, wg_blk, preferred_element_type=jnp.float32)
            u = jnp.dot(xv_my, wu_blk, preferred_element_type=jnp.float32)
            h = (g * (u * jax.nn.sigmoid(u))).astype(jnp.bfloat16)
            return acc + jnp.dot(h, wd_blk, preferred_element_type=jnp.float32)

        start_block(0)
        acc0 = lax.fori_loop(
            0, NB, stream_step, jnp.zeros((CH, M), jnp.float32)
        )
        out_ref[my_row, :] = acc0.astype(jnp.bfloat16)

        def compute_chunk(c):
            row = pl.ds(c * CH, CH)
            xv = x_ref[row, :]
            g = jnp.dot(xv, wgb[...], preferred_element_type=jnp.float32)
            u = jnp.dot(xv, wub[...], preferred_element_type=jnp.float32)
            h = (g * (u * jax.nn.sigmoid(u))).astype(jnp.bfloat16)
            out_ref[row, :] = jnp.dot(
                h, wdb[...], preferred_element_type=jnp.float32
            ).astype(jnp.bfloat16)

        barrier_sem = pltpu.get_barrier_semaphore()
        for nbr in (left, right):
            pl.semaphore_signal(barrier_sem, inc=1, device_id=(nbr,),
                                device_id_type=pl.DeviceIdType.MESH)
        pl.semaphore_wait(barrier_sem, 2)

        def rs_step(s, _):
            send_c = (my - s) % N_DEV
            rdma = pltpu.make_async_remote_copy(
                src_ref=out_ref.at[pl.ds(send_c * CH, CH), :],
                dst_ref=recv_buf.at[s % 2],
                send_sem=rs_send_sems.at[s],
                recv_sem=rs_recv_sems.at[s],
                device_id=(right,),
                device_id_type=pl.DeviceIdType.MESH,
            )
            rdma.start()
            rc = (my - s - 1) % N_DEV
            compute_chunk(rc)
            rdma.wait_recv()
            row = pl.ds(rc * CH, CH)
            out_ref[row, :] = out_ref[row, :] + recv_buf[s % 2]
            rdma.wait_send()
            return _

        lax.fori_loop(0, N_DEV - 1, rs_step, None)

        own_row = pl.ds(((my + 1) % N_DEV) * CH, CH)
        z_rdma = pltpu.make_async_remote_copy(
            src_ref=out_ref.at[own_row, :],
            dst_ref=out_ref.at[own_row, :],
            send_sem=ag_send_sems.at[6],
            recv_sem=ag_recv_sems.at[6],
            device_id=((my + 4) % N_DEV,),
            device_id_type=pl.DeviceIdType.MESH,
        )
        z_rdma.start()
        ag_rdmas = []
        for t in range(3):
            if t > 0:
                ag_rdmas[2 * t - 2].wait_recv()
                ag_rdmas[2 * t - 1].wait_recv()
            rowr = pl.ds(((my + 1 - t) % N_DEV) * CH, CH)
            r_rdma = pltpu.make_async_remote_copy(
                src_ref=out_ref.at[rowr, :],
                dst_ref=out_ref.at[rowr, :],
                send_sem=ag_send_sems.at[t],
                recv_sem=ag_recv_sems.at[t],
                device_id=(right,),
                device_id_type=pl.DeviceIdType.MESH,
            )
            r_rdma.start()
            rowl = pl.ds(((my + 1 + t) % N_DEV) * CH, CH)
            l_rdma = pltpu.make_async_remote_copy(
                src_ref=out_ref.at[rowl, :],
                dst_ref=out_ref.at[rowl, :],
                send_sem=ag_send_sems.at[3 + t],
                recv_sem=ag_recv_sems.at[3 + t],
                device_id=(left,),
                device_id_type=pl.DeviceIdType.MESH,
            )
            l_rdma.start()
            ag_rdmas += [r_rdma, l_rdma]
        ag_rdmas[4].wait_recv()
        ag_rdmas[5].wait_recv()
        z_rdma.wait_recv()
        for rdma in ag_rdmas + [z_rdma]:
            rdma.wait_send()

    return pl.pallas_call(
        body,
        out_shape=jax.ShapeDtypeStruct((M, M), jnp.bfloat16),
        in_specs=[
            pl.BlockSpec(memory_space=pltpu.VMEM),
            pl.BlockSpec(memory_space=pl.ANY),
            pl.BlockSpec(memory_space=pl.ANY),
            pl.BlockSpec(memory_space=pl.ANY),
        ],
        out_specs=pl.BlockSpec(memory_space=pltpu.VMEM),
        scratch_shapes=[
            pltpu.VMEM((M, H_SH), jnp.bfloat16),
            pltpu.VMEM((M, H_SH), jnp.bfloat16),
            pltpu.VMEM((H_SH, M), jnp.bfloat16),
            pltpu.VMEM((2, M, BW), jnp.float32),
            pltpu.VMEM((2, M, BW), jnp.float32),
            pltpu.VMEM((2, BW, M), jnp.float32),
            pltpu.SemaphoreType.DMA((3, 2)),
            pltpu.VMEM((2, CH, M), jnp.bfloat16),
            pltpu.SemaphoreType.DMA((N_DEV - 1,)),
            pltpu.SemaphoreType.DMA((N_DEV - 1,)),
            pltpu.SemaphoreType.DMA((N_DEV - 1,)),
            pltpu.SemaphoreType.DMA((N_DEV - 1,)),
        ],
        compiler_params=pltpu.CompilerParams(
            collective_id=0,
            vmem_limit_bytes=63 * 1024 * 1024,
        ),
    )(xb, Wg, Wu, Wd)
